# baseline (speedup 1.0000x reference)
import functools

import jax
import jax.numpy as jnp
from jax import lax
from jax.experimental import pallas as pl
from jax.experimental.pallas import tpu as pltpu
from jax.experimental.pallas import tpu_sc as plsc

N = 10000
E = 320000
D = 128
NC = 2
NS = 16
NW = NC * NS
K = 128
NPAD = 10240
RPT = NPAD // NS
EPT = NPAD
EPAD = NW * EPT
NCHUNK = EPT // K
NBUF = 2

ROWBLK = 1000
NB = N // ROWBLK


def _sc_segment_sums(h_user, h_item, src0, dst0, src1, dst1):
  zrows = jnp.zeros((RPT, D), jnp.float32)
  zhist = jnp.zeros((NPAD,), jnp.float32)
  pad_s = jnp.zeros((EPAD - E,), src0.dtype)
  pad_d = jnp.full((EPAD - E,), NPAD - 1, dst0.dtype)
  src0 = jnp.concatenate([src0, pad_s])
  dst0 = jnp.concatenate([dst0, pad_d])
  src1 = jnp.concatenate([src1, pad_s])
  dst1 = jnp.concatenate([dst1, pad_d])

  mesh = plsc.VectorSubcoreMesh(core_axis_name="c", subcore_axis_name="s")

  @functools.partial(
      pl.kernel,
      out_type=(
          jax.ShapeDtypeStruct((2 * NC * NPAD, D), jnp.float32),
          jax.ShapeDtypeStruct((2 * NW * NPAD,), jnp.float32),
      ),
      mesh=mesh,
      compiler_params=pltpu.CompilerParams(needs_layout_passes=False),
      scratch_types=[
          pltpu.VMEM((NBUF, K), jnp.int32),
          pltpu.VMEM((NBUF, K), jnp.int32),
          pltpu.VMEM((NBUF, K, D), jnp.float32),
          pltpu.VMEM((NPAD,), jnp.float32),
          pltpu.VMEM_SHARED((NPAD, D), jnp.float32),
          pltpu.SemaphoreType.DMA,
          pltpu.SemaphoreType.DMA,
      ],
  )
  def seg(hu, hi, s0, d0, s1, d1, zr, zh, agg_out, cnt_out,
          idx_s, idx_d, rows, hist, acc, sem0, sem1):
    c = lax.axis_index("c")
    s = lax.axis_index("s")
    wid = c * NS + s
    rowbase = pl.multiple_of(s * RPT, 8)
    ebase = wid * EPT
    ones16 = jnp.ones((16,), jnp.float32)
    sems = (sem0, sem1)

    def zero_owned():
      pltpu.sync_copy(zr, acc.at[pl.ds(rowbase, RPT)])
      pltpu.sync_copy(zh, hist)

    zero_owned()
    plsc.subcore_barrier()

    def do_rel(rel, table, src_hbm, dst_hbm):
      def load_and_fire(ch, b):
        eoff = pl.multiple_of(ebase + ch * K, 8)
        pltpu.sync_copy(src_hbm.at[pl.ds(eoff, K)], idx_s.at[b])
        pltpu.sync_copy(dst_hbm.at[pl.ds(eoff, K)], idx_d.at[b])
        pltpu.async_copy(table.at[idx_s.at[b]], rows.at[b], sems[b])

      for b in range(NBUF):
        load_and_fire(b, b)

      def outer(g, carry):
        for b in range(NBUF):
          ch = NBUF * g + b
          pltpu.make_async_copy(table.at[pl.ds(0, K)], rows.at[b],
                                sems[b]).wait()
          pltpu.sync_copy(rows.at[b], acc.at[idx_d.at[b]], add=True)
          for t in range(K // 16):
            plsc.addupdate_scatter(hist, [idx_d[b, pl.ds(t * 16, 16)]],
                                   ones16)
          @pl.when(ch + NBUF < NCHUNK)
          def _():
            load_and_fire(ch + NBUF, b)
        return carry
      lax.fori_loop(0, NCHUNK // NBUF, outer, 0)
      plsc.subcore_barrier()
      obase = pl.multiple_of((rel * NC + c) * NPAD + rowbase, 8)
      pltpu.sync_copy(acc.at[pl.ds(rowbase, RPT)],
                      agg_out.at[pl.ds(obase, RPT)])
      hbase = pl.multiple_of((rel * NW + wid) * NPAD, 8)
      pltpu.sync_copy(hist, cnt_out.at[pl.ds(hbase, NPAD)])

    do_rel(0, hu, s0, d0)
    zero_owned()
    plsc.subcore_barrier()
    do_rel(1, hi, s1, d1)

  agg, cnt = seg(h_user, h_item, src0, dst0, src1, dst1, zrows, zhist)
  return (agg.reshape(2, NC, NPAD, D), cnt.reshape(2, NW, NPAD))


def _lin_body(x_ref, w_ref, b_ref, o_ref):
  o_ref[...] = lax.dot_general(
      x_ref[...], w_ref[...], (((1,), (1,)), ((), ())),
      preferred_element_type=jnp.float32) + b_ref[...]


def _input_transform(x, w, b):
  return pl.pallas_call(
      _lin_body,
      grid=(NB,),
      in_specs=[
          pl.BlockSpec((ROWBLK, D), lambda i: (i, 0)),
          pl.BlockSpec((D, D), lambda i: (0, 0)),
          pl.BlockSpec((1, D), lambda i: (0, 0)),
      ],
      out_specs=pl.BlockSpec((ROWBLK, D), lambda i: (i, 0)),
      out_shape=jax.ShapeDtypeStruct((N, D), jnp.float32),
  )(x, w, b.reshape(1, D))


def _post_body(hd_ref, a0_ref, a1_ref, c_ref,
               wl_ref, bl_ref, wr_ref, g_ref, be_ref, o_ref):
  hd = hd_ref[...]
  agg = a0_ref[0] + a1_ref[0]
  cnt = jnp.sum(c_ref[...], axis=1, keepdims=True)
  mean = agg / jnp.maximum(cnt, 1.0)
  out = (lax.dot_general(mean, wl_ref[...], (((1,), (1,)), ((), ())),
                         preferred_element_type=jnp.float32)
         + bl_ref[...]
         + lax.dot_general(hd, wr_ref[...], (((1,), (1,)), ((), ())),
                           preferred_element_type=jnp.float32))
  nrm = jnp.sqrt(jnp.sum(out * out, axis=-1, keepdims=True))
  conv = out / jnp.maximum(nrm, 1e-12)
  y = hd + conv
  mu = jnp.mean(y, axis=-1, keepdims=True)
  var = jnp.mean((y - mu) ** 2, axis=-1, keepdims=True)
  o_ref[...] = (y - mu) / jnp.sqrt(var + 1e-5) * g_ref[...] + be_ref[...]


def _post(hd, agg_pair, cnt_hists, wl, bl, wr, g, be):
  return pl.pallas_call(
      _post_body,
      grid=(NB,),
      in_specs=[
          pl.BlockSpec((ROWBLK, D), lambda i: (i, 0)),
          pl.BlockSpec((1, ROWBLK, D), lambda i: (0, i, 0)),
          pl.BlockSpec((1, ROWBLK, D), lambda i: (1, i, 0)),
          pl.BlockSpec((ROWBLK, NW), lambda i: (i, 0)),
          pl.BlockSpec((D, D), lambda i: (0, 0)),
          pl.BlockSpec((1, D), lambda i: (0, 0)),
          pl.BlockSpec((D, D), lambda i: (0, 0)),
          pl.BlockSpec((1, D), lambda i: (0, 0)),
          pl.BlockSpec((1, D), lambda i: (0, 0)),
      ],
      out_specs=pl.BlockSpec((ROWBLK, D), lambda i: (i, 0)),
      out_shape=jax.ShapeDtypeStruct((N, D), jnp.float32),
  )(hd, agg_pair, agg_pair, cnt_hists, wl, bl.reshape(1, D), wr,
    g.reshape(1, D), be.reshape(1, D))


def kernel(x_user, x_item, edge_index_user_item, edge_index_item_user,
           W_user, b_user, W_item, b_item,
           Wl_ui, bl_ui, Wr_ui, Wl_iu, bl_iu, Wr_iu,
           ln_g_user, ln_b_user, ln_g_item, ln_b_item):
  h_user = _input_transform(x_user, W_user, b_user)
  h_item = _input_transform(x_item, W_item, b_item)

  agg, cnt = _sc_segment_sums(
      h_user, h_item,
      edge_index_user_item[0], edge_index_user_item[1],
      edge_index_item_user[0], edge_index_item_user[1])

  cnt_t = jnp.transpose(cnt, (0, 2, 1))
  out_item = _post(h_item, agg[0], cnt_t[0], Wl_ui, bl_ui, Wr_ui,
                   ln_g_item, ln_b_item)
  out_user = _post(h_user, agg[1], cnt_t[1], Wl_iu, bl_iu, Wr_iu,
                   ln_g_user, ln_b_user)
  return (out_user, out_item)


# speedup vs baseline: 1.1332x; 1.1332x over previous
import functools

import jax
import jax.numpy as jnp
from jax import lax
from jax.experimental import pallas as pl
from jax.experimental.pallas import tpu as pltpu
from jax.experimental.pallas import tpu_sc as plsc

N = 10000
E = 320000
D = 128
NC = 2
NS = 16
NW = NC * NS
K = 128
NPAD = 10240
RPT = NPAD // NS
EPT = NPAD
EPAD = NW * EPT
NCHUNK = EPT // K
NBUF = 2

ROWBLK = 1000
NB = N // ROWBLK


def _sc_segment_sums(h_user, h_item, src0, dst0, src1, dst1):
  zrows = jnp.zeros((RPT, D), jnp.float32)
  zhist = jnp.zeros((NPAD,), jnp.float32)
  pad_s = jnp.zeros((EPAD - E,), src0.dtype)
  pad_d = (N + jnp.arange(EPAD - E, dtype=dst0.dtype) % (NPAD - N))
  src0 = jnp.concatenate([src0, pad_s])
  dst0 = jnp.concatenate([dst0, pad_d])
  src1 = jnp.concatenate([src1, pad_s])
  dst1 = jnp.concatenate([dst1, pad_d])

  mesh = plsc.VectorSubcoreMesh(core_axis_name="c", subcore_axis_name="s")

  @functools.partial(
      pl.kernel,
      out_type=(
          jax.ShapeDtypeStruct((2 * NC * NPAD, D), jnp.float32),
          jax.ShapeDtypeStruct((2 * NW * NPAD,), jnp.float32),
      ),
      mesh=mesh,
      compiler_params=pltpu.CompilerParams(needs_layout_passes=False),
      scratch_types=[
          pltpu.VMEM((NBUF, K), jnp.int32),
          pltpu.VMEM((NBUF, K), jnp.int32),
          pltpu.VMEM((NBUF, K, D), jnp.float32),
          pltpu.VMEM((NPAD,), jnp.float32),
          pltpu.VMEM_SHARED((NPAD, D), jnp.float32),
          pltpu.SemaphoreType.DMA,
          pltpu.SemaphoreType.DMA,
      ],
  )
  def seg(hu, hi, s0, d0, s1, d1, zr, zh, agg_out, cnt_out,
          idx_s, idx_d, rows, hist, acc, sem0, sem1):
    c = lax.axis_index("c")
    s = lax.axis_index("s")
    wid = c * NS + s
    rowbase = pl.multiple_of(s * RPT, 8)
    ebase = wid * EPT
    ones16 = jnp.ones((16,), jnp.float32)
    sems = (sem0, sem1)

    def zero_owned():
      pltpu.sync_copy(zr, acc.at[pl.ds(rowbase, RPT)])
      pltpu.sync_copy(zh, hist)

    zero_owned()
    plsc.subcore_barrier()

    def do_rel(rel, table, src_hbm, dst_hbm):
      def load_and_fire(ch, b):
        eoff = pl.multiple_of(ebase + ch * K, 8)
        pltpu.sync_copy(src_hbm.at[pl.ds(eoff, K)], idx_s.at[b])
        pltpu.sync_copy(dst_hbm.at[pl.ds(eoff, K)], idx_d.at[b])
        pltpu.async_copy(table.at[idx_s.at[b]], rows.at[b], sems[b])

      for b in range(NBUF):
        load_and_fire(b, b)

      def outer(g, carry):
        for b in range(NBUF):
          ch = NBUF * g + b
          pltpu.make_async_copy(table.at[pl.ds(0, K)], rows.at[b],
                                sems[b]).wait()
          pltpu.sync_copy(rows.at[b], acc.at[idx_d.at[b]], add=True)
          for t in range(K // 16):
            plsc.addupdate_scatter(hist, [idx_d[b, pl.ds(t * 16, 16)]],
                                   ones16)
          @pl.when(ch + NBUF < NCHUNK)
          def _():
            load_and_fire(ch + NBUF, b)
        return carry
      lax.fori_loop(0, NCHUNK // NBUF, outer, 0)
      plsc.subcore_barrier()
      obase = pl.multiple_of((rel * NC + c) * NPAD + rowbase, 8)
      pltpu.sync_copy(acc.at[pl.ds(rowbase, RPT)],
                      agg_out.at[pl.ds(obase, RPT)])
      hbase = pl.multiple_of((rel * NW + wid) * NPAD, 8)
      pltpu.sync_copy(hist, cnt_out.at[pl.ds(hbase, NPAD)])

    do_rel(0, hu, s0, d0)
    zero_owned()
    plsc.subcore_barrier()
    do_rel(1, hi, s1, d1)

  agg, cnt = seg(h_user, h_item, src0, dst0, src1, dst1, zrows, zhist)
  return (agg.reshape(2, NC, NPAD, D), cnt.reshape(2, NW, NPAD))


def _lin_body(x_ref, w_ref, b_ref, o_ref):
  o_ref[...] = lax.dot_general(
      x_ref[...], w_ref[...], (((1,), (1,)), ((), ())),
      preferred_element_type=jnp.float32) + b_ref[...]


def _input_transform(x, w, b):
  return pl.pallas_call(
      _lin_body,
      grid=(NB,),
      in_specs=[
          pl.BlockSpec((ROWBLK, D), lambda i: (i, 0)),
          pl.BlockSpec((D, D), lambda i: (0, 0)),
          pl.BlockSpec((1, D), lambda i: (0, 0)),
      ],
      out_specs=pl.BlockSpec((ROWBLK, D), lambda i: (i, 0)),
      out_shape=jax.ShapeDtypeStruct((N, D), jnp.float32),
  )(x, w, b.reshape(1, D))


def _post_body(hd_ref, a0_ref, a1_ref, c_ref,
               wl_ref, bl_ref, wr_ref, g_ref, be_ref, o_ref):
  hd = hd_ref[...]
  agg = a0_ref[0] + a1_ref[0]
  cnt = jnp.sum(c_ref[...], axis=1, keepdims=True)
  mean = agg / jnp.maximum(cnt, 1.0)
  out = (lax.dot_general(mean, wl_ref[...], (((1,), (1,)), ((), ())),
                         preferred_element_type=jnp.float32)
         + bl_ref[...]
         + lax.dot_general(hd, wr_ref[...], (((1,), (1,)), ((), ())),
                           preferred_element_type=jnp.float32))
  nrm = jnp.sqrt(jnp.sum(out * out, axis=-1, keepdims=True))
  conv = out / jnp.maximum(nrm, 1e-12)
  y = hd + conv
  mu = jnp.mean(y, axis=-1, keepdims=True)
  var = jnp.mean((y - mu) ** 2, axis=-1, keepdims=True)
  o_ref[...] = (y - mu) / jnp.sqrt(var + 1e-5) * g_ref[...] + be_ref[...]


def _post(hd, agg_pair, cnt_hists, wl, bl, wr, g, be):
  return pl.pallas_call(
      _post_body,
      grid=(NB,),
      in_specs=[
          pl.BlockSpec((ROWBLK, D), lambda i: (i, 0)),
          pl.BlockSpec((1, ROWBLK, D), lambda i: (0, i, 0)),
          pl.BlockSpec((1, ROWBLK, D), lambda i: (1, i, 0)),
          pl.BlockSpec((ROWBLK, NW), lambda i: (i, 0)),
          pl.BlockSpec((D, D), lambda i: (0, 0)),
          pl.BlockSpec((1, D), lambda i: (0, 0)),
          pl.BlockSpec((D, D), lambda i: (0, 0)),
          pl.BlockSpec((1, D), lambda i: (0, 0)),
          pl.BlockSpec((1, D), lambda i: (0, 0)),
      ],
      out_specs=pl.BlockSpec((ROWBLK, D), lambda i: (i, 0)),
      out_shape=jax.ShapeDtypeStruct((N, D), jnp.float32),
  )(hd, agg_pair, agg_pair, cnt_hists, wl, bl.reshape(1, D), wr,
    g.reshape(1, D), be.reshape(1, D))


def kernel(x_user, x_item, edge_index_user_item, edge_index_item_user,
           W_user, b_user, W_item, b_item,
           Wl_ui, bl_ui, Wr_ui, Wl_iu, bl_iu, Wr_iu,
           ln_g_user, ln_b_user, ln_g_item, ln_b_item):
  h_user = _input_transform(x_user, W_user, b_user)
  h_item = _input_transform(x_item, W_item, b_item)

  agg, cnt = _sc_segment_sums(
      h_user, h_item,
      edge_index_user_item[0], edge_index_user_item[1],
      edge_index_item_user[0], edge_index_item_user[1])

  cnt_t = jnp.transpose(cnt, (0, 2, 1))
  out_item = _post(h_item, agg[0], cnt_t[0], Wl_ui, bl_ui, Wr_ui,
                   ln_g_item, ln_b_item)
  out_user = _post(h_user, agg[1], cnt_t[1], Wl_iu, bl_iu, Wr_iu,
                   ln_g_user, ln_b_user)
  return (out_user, out_item)


# speedup vs baseline: 2.7124x; 2.3936x over previous
import functools

import jax
import jax.numpy as jnp
from jax import lax
from jax.experimental import pallas as pl
from jax.experimental.pallas import tpu as pltpu
from jax.experimental.pallas import tpu_sc as plsc

N = 10000
E = 320000
D = 128
NC = 2
NS = 16
NW = NC * NS
K = 128
NPAD = 10240
RPT = NPAD // NS
EPT = NPAD
EPAD = NW * EPT
NCHUNK = EPT // K
NBUF = 2

ROWBLK = 1000
NB = N // ROWBLK


def _sc_segment_sums(h_user, h_item, src0, dst0, src1, dst1):
  zrows = jnp.zeros((RPT, D), jnp.float32)
  zhist = jnp.zeros((NPAD,), jnp.float32)
  pad_s = jnp.arange(EPAD - E, dtype=src0.dtype) % N
  pad_d = (N + jnp.arange(EPAD - E, dtype=dst0.dtype) % (NPAD - N))
  src0 = jnp.concatenate([src0, pad_s])
  dst0 = jnp.concatenate([dst0, pad_d])
  src1 = jnp.concatenate([src1, pad_s])
  dst1 = jnp.concatenate([dst1, pad_d])

  mesh = plsc.VectorSubcoreMesh(core_axis_name="c", subcore_axis_name="s")

  @functools.partial(
      pl.kernel,
      out_type=(
          jax.ShapeDtypeStruct((2 * NC * NPAD, D), jnp.float32),
          jax.ShapeDtypeStruct((2 * NW * NPAD,), jnp.float32),
      ),
      mesh=mesh,
      compiler_params=pltpu.CompilerParams(needs_layout_passes=False),
      scratch_types=[
          pltpu.VMEM((NBUF, K), jnp.int32),
          pltpu.VMEM((NBUF, K), jnp.int32),
          pltpu.VMEM((NBUF, K, D), jnp.float32),
          pltpu.VMEM((NPAD,), jnp.float32),
          pltpu.VMEM_SHARED((NPAD, D), jnp.float32),
          pltpu.SemaphoreType.DMA,
          pltpu.SemaphoreType.DMA,
      ],
  )
  def seg(hu, hi, s0, d0, s1, d1, zr, zh, agg_out, cnt_out,
          idx_s, idx_d, rows, hist, acc, sem0, sem1):
    c = lax.axis_index("c")
    s = lax.axis_index("s")
    wid = c * NS + s
    rowbase = pl.multiple_of(s * RPT, 8)
    ebase = wid * EPT
    ones16 = jnp.ones((16,), jnp.float32)
    sems = (sem0, sem1)

    def zero_owned():
      pltpu.sync_copy(zr, acc.at[pl.ds(rowbase, RPT)])
      pltpu.sync_copy(zh, hist)

    zero_owned()
    plsc.subcore_barrier()

    def do_rel(rel, table, src_hbm, dst_hbm):
      def load_and_fire(ch, b):
        eoff = pl.multiple_of(ebase + ch * K, 8)
        pltpu.sync_copy(src_hbm.at[pl.ds(eoff, K)], idx_s.at[b])
        pltpu.sync_copy(dst_hbm.at[pl.ds(eoff, K)], idx_d.at[b])
        pltpu.async_copy(table.at[idx_s.at[b]], rows.at[b], sems[b])

      for b in range(NBUF):
        load_and_fire(b, b)

      def outer(g, carry):
        for b in range(NBUF):
          ch = NBUF * g + b
          pltpu.make_async_copy(table.at[pl.ds(0, K)], rows.at[b],
                                sems[b]).wait()
          pltpu.sync_copy(rows.at[b], acc.at[idx_d.at[b]], add=True)
          for t in range(K // 16):
            plsc.addupdate_scatter(hist, [idx_d[b, pl.ds(t * 16, 16)]],
                                   ones16)
          @pl.when(ch + NBUF < NCHUNK)
          def _():
            load_and_fire(ch + NBUF, b)
        return carry
      lax.fori_loop(0, NCHUNK // NBUF, outer, 0)
      plsc.subcore_barrier()
      obase = pl.multiple_of((rel * NC + c) * NPAD + rowbase, 8)
      pltpu.sync_copy(acc.at[pl.ds(rowbase, RPT)],
                      agg_out.at[pl.ds(obase, RPT)])
      hbase = pl.multiple_of((rel * NW + wid) * NPAD, 8)
      pltpu.sync_copy(hist, cnt_out.at[pl.ds(hbase, NPAD)])

    do_rel(0, hu, s0, d0)
    zero_owned()
    plsc.subcore_barrier()
    do_rel(1, hi, s1, d1)

  agg, cnt = seg(h_user, h_item, src0, dst0, src1, dst1, zrows, zhist)
  return (agg.reshape(2, NC, NPAD, D), cnt.reshape(2, NW, NPAD))


def _lin_body(x_ref, w_ref, b_ref, o_ref):
  o_ref[...] = lax.dot_general(
      x_ref[...], w_ref[...], (((1,), (1,)), ((), ())),
      preferred_element_type=jnp.float32) + b_ref[...]


def _input_transform(x, w, b):
  return pl.pallas_call(
      _lin_body,
      grid=(NB,),
      in_specs=[
          pl.BlockSpec((ROWBLK, D), lambda i: (i, 0)),
          pl.BlockSpec((D, D), lambda i: (0, 0)),
          pl.BlockSpec((1, D), lambda i: (0, 0)),
      ],
      out_specs=pl.BlockSpec((ROWBLK, D), lambda i: (i, 0)),
      out_shape=jax.ShapeDtypeStruct((N, D), jnp.float32),
  )(x, w, b.reshape(1, D))


def _post_body(hd_ref, a0_ref, a1_ref, c_ref,
               wl_ref, bl_ref, wr_ref, g_ref, be_ref, o_ref):
  hd = hd_ref[...]
  agg = a0_ref[0] + a1_ref[0]
  cnt = jnp.sum(c_ref[...], axis=1, keepdims=True)
  mean = agg / jnp.maximum(cnt, 1.0)
  out = (lax.dot_general(mean, wl_ref[...], (((1,), (1,)), ((), ())),
                         preferred_element_type=jnp.float32)
         + bl_ref[...]
         + lax.dot_general(hd, wr_ref[...], (((1,), (1,)), ((), ())),
                           preferred_element_type=jnp.float32))
  nrm = jnp.sqrt(jnp.sum(out * out, axis=-1, keepdims=True))
  conv = out / jnp.maximum(nrm, 1e-12)
  y = hd + conv
  mu = jnp.mean(y, axis=-1, keepdims=True)
  var = jnp.mean((y - mu) ** 2, axis=-1, keepdims=True)
  o_ref[...] = (y - mu) / jnp.sqrt(var + 1e-5) * g_ref[...] + be_ref[...]


def _post(hd, agg_pair, cnt_hists, wl, bl, wr, g, be):
  return pl.pallas_call(
      _post_body,
      grid=(NB,),
      in_specs=[
          pl.BlockSpec((ROWBLK, D), lambda i: (i, 0)),
          pl.BlockSpec((1, ROWBLK, D), lambda i: (0, i, 0)),
          pl.BlockSpec((1, ROWBLK, D), lambda i: (1, i, 0)),
          pl.BlockSpec((ROWBLK, NW), lambda i: (i, 0)),
          pl.BlockSpec((D, D), lambda i: (0, 0)),
          pl.BlockSpec((1, D), lambda i: (0, 0)),
          pl.BlockSpec((D, D), lambda i: (0, 0)),
          pl.BlockSpec((1, D), lambda i: (0, 0)),
          pl.BlockSpec((1, D), lambda i: (0, 0)),
      ],
      out_specs=pl.BlockSpec((ROWBLK, D), lambda i: (i, 0)),
      out_shape=jax.ShapeDtypeStruct((N, D), jnp.float32),
  )(hd, agg_pair, agg_pair, cnt_hists, wl, bl.reshape(1, D), wr,
    g.reshape(1, D), be.reshape(1, D))


def kernel(x_user, x_item, edge_index_user_item, edge_index_item_user,
           W_user, b_user, W_item, b_item,
           Wl_ui, bl_ui, Wr_ui, Wl_iu, bl_iu, Wr_iu,
           ln_g_user, ln_b_user, ln_g_item, ln_b_item):
  h_user = _input_transform(x_user, W_user, b_user)
  h_item = _input_transform(x_item, W_item, b_item)

  agg, cnt = _sc_segment_sums(
      h_user, h_item,
      edge_index_user_item[0], edge_index_user_item[1],
      edge_index_item_user[0], edge_index_item_user[1])

  cnt_t = jnp.transpose(cnt, (0, 2, 1))
  out_item = _post(h_item, agg[0], cnt_t[0], Wl_ui, bl_ui, Wr_ui,
                   ln_g_item, ln_b_item)
  out_user = _post(h_user, agg[1], cnt_t[1], Wl_iu, bl_iu, Wr_iu,
                   ln_g_user, ln_b_user)
  return (out_user, out_item)


# speedup vs baseline: 3.4248x; 1.2627x over previous
import functools

import jax
import jax.numpy as jnp
from jax import lax
from jax.experimental import pallas as pl
from jax.experimental.pallas import tpu as pltpu
from jax.experimental.pallas import tpu_sc as plsc

N = 10000
E = 320000
D = 128
NC = 2
NS = 16
NW = NC * NS
K = 128
NPAD = 10240
RPT = NPAD // NS
EPT = NPAD
EPAD = NW * EPT
NCHUNK = EPT // K
NBUF = 2
NI = 8
ILA = 6

ROWBLK = 1000
NB = N // ROWBLK


def _sc_segment_sums(h_user, h_item, src0, dst0, src1, dst1):
  zrows = jnp.zeros((RPT, D), jnp.float32)
  zhist = jnp.zeros((NPAD,), jnp.float32)
  pad_s = jnp.arange(EPAD - E, dtype=src0.dtype) % N
  pad_d = (N + jnp.arange(EPAD - E, dtype=dst0.dtype) % (NPAD - N))
  src0 = jnp.concatenate([src0, pad_s])
  dst0 = jnp.concatenate([dst0, pad_d])
  src1 = jnp.concatenate([src1, pad_s])
  dst1 = jnp.concatenate([dst1, pad_d])

  mesh = plsc.VectorSubcoreMesh(core_axis_name="c", subcore_axis_name="s")

  @functools.partial(
      pl.kernel,
      out_type=(
          jax.ShapeDtypeStruct((2 * NC * NPAD, D), jnp.float32),
          jax.ShapeDtypeStruct((2 * NW * NPAD,), jnp.float32),
      ),
      mesh=mesh,
      compiler_params=pltpu.CompilerParams(needs_layout_passes=False),
      scratch_types=[
          pltpu.VMEM((NI, K), jnp.int32),
          pltpu.VMEM((NI, K), jnp.int32),
          pltpu.VMEM((NBUF, K, D), jnp.float32),
          pltpu.VMEM((NPAD,), jnp.float32),
          pltpu.VMEM_SHARED((NPAD, D), jnp.float32),
          [pltpu.SemaphoreType.DMA] * NI,
          [pltpu.SemaphoreType.DMA] * NBUF,
          [pltpu.SemaphoreType.DMA] * NBUF,
      ],
  )
  def seg(hu, hi, s0, d0, s1, d1, zr, zh, agg_out, cnt_out,
          idx_s, idx_d, rows, hist, acc, sem_i, sem_g, sem_s):
    c = lax.axis_index("c")
    s = lax.axis_index("s")
    wid = c * NS + s
    rowbase = pl.multiple_of(s * RPT, 8)
    ebase = wid * EPT
    ones16 = jnp.ones((16,), jnp.float32)

    def zero_owned():
      pltpu.sync_copy(zr, acc.at[pl.ds(rowbase, RPT)])
      pltpu.sync_copy(zh, hist)

    zero_owned()
    plsc.subcore_barrier()

    def do_rel(rel, table, src_hbm, dst_hbm):
      def fire_idx(ch, bi):
        eoff = pl.multiple_of(ebase + ch * K, 8)
        pltpu.async_copy(src_hbm.at[pl.ds(eoff, K)], idx_s.at[bi], sem_i[bi])
        pltpu.async_copy(dst_hbm.at[pl.ds(eoff, K)], idx_d.at[bi], sem_i[bi])

      def wait_idx(bi):
        pltpu.make_async_copy(src_hbm.at[pl.ds(0, K)], idx_s.at[bi],
                              sem_i[bi]).wait()
        pltpu.make_async_copy(src_hbm.at[pl.ds(0, K)], idx_d.at[bi],
                              sem_i[bi]).wait()

      def fire_gather(bi, b):
        pltpu.async_copy(table.at[idx_s.at[bi]], rows.at[b], sem_g[b])

      for ch in range(ILA):
        fire_idx(ch, ch % NI)
      for ch in range(NBUF):
        wait_idx(ch % NI)
        fire_gather(ch % NI, ch % NBUF)

      def outer(g, carry):
        for u in range(NI):
          ch = NI * g + u
          b = u % NBUF
          bi = u
          @pl.when(ch + ILA < NCHUNK)
          def _():
            fire_idx(ch + ILA, (u + ILA) % NI)
          pltpu.make_async_copy(table.at[pl.ds(0, K)], rows.at[b],
                                sem_g[b]).wait()
          pltpu.async_copy(rows.at[b], acc.at[idx_d.at[bi]], sem_s[b],
                           add=True)
          for t in range(K // 16):
            plsc.addupdate_scatter(hist, [idx_d[bi, pl.ds(t * 16, 16)]],
                                   ones16)
          @pl.when(ch + NBUF < NCHUNK)
          def _():
            pltpu.make_async_copy(table.at[pl.ds(0, K)], rows.at[b],
                                  sem_s[b]).wait()
            wait_idx((u + NBUF) % NI)
            fire_gather((u + NBUF) % NI, b)
        return carry
      lax.fori_loop(0, NCHUNK // NI, outer, 0)
      for b in range(NBUF):
        pltpu.make_async_copy(table.at[pl.ds(0, K)], rows.at[b],
                              sem_s[b]).wait()
      plsc.subcore_barrier()
      obase = pl.multiple_of((rel * NC + c) * NPAD + rowbase, 8)
      pltpu.sync_copy(acc.at[pl.ds(rowbase, RPT)],
                      agg_out.at[pl.ds(obase, RPT)])
      hbase = pl.multiple_of((rel * NW + wid) * NPAD, 8)
      pltpu.sync_copy(hist, cnt_out.at[pl.ds(hbase, NPAD)])

    do_rel(0, hu, s0, d0)
    zero_owned()
    plsc.subcore_barrier()
    do_rel(1, hi, s1, d1)

  agg, cnt = seg(h_user, h_item, src0, dst0, src1, dst1, zrows, zhist)
  return (agg.reshape(2, NC, NPAD, D), cnt.reshape(2, NW, NPAD))


def _lin_body(x_ref, w_ref, b_ref, o_ref):
  o_ref[...] = lax.dot_general(
      x_ref[...], w_ref[...], (((1,), (1,)), ((), ())),
      preferred_element_type=jnp.float32) + b_ref[...]


def _input_transform(x, w, b):
  return pl.pallas_call(
      _lin_body,
      grid=(NB,),
      in_specs=[
          pl.BlockSpec((ROWBLK, D), lambda i: (i, 0)),
          pl.BlockSpec((D, D), lambda i: (0, 0)),
          pl.BlockSpec((1, D), lambda i: (0, 0)),
      ],
      out_specs=pl.BlockSpec((ROWBLK, D), lambda i: (i, 0)),
      out_shape=jax.ShapeDtypeStruct((N, D), jnp.float32),
  )(x, w, b.reshape(1, D))


def _post_body(hd_ref, a0_ref, a1_ref, c_ref,
               wl_ref, bl_ref, wr_ref, g_ref, be_ref, o_ref):
  hd = hd_ref[...]
  agg = a0_ref[0] + a1_ref[0]
  cnt = jnp.sum(c_ref[...], axis=1, keepdims=True)
  mean = agg / jnp.maximum(cnt, 1.0)
  out = (lax.dot_general(mean, wl_ref[...], (((1,), (1,)), ((), ())),
                         preferred_element_type=jnp.float32)
         + bl_ref[...]
         + lax.dot_general(hd, wr_ref[...], (((1,), (1,)), ((), ())),
                           preferred_element_type=jnp.float32))
  nrm = jnp.sqrt(jnp.sum(out * out, axis=-1, keepdims=True))
  conv = out / jnp.maximum(nrm, 1e-12)
  y = hd + conv
  mu = jnp.mean(y, axis=-1, keepdims=True)
  var = jnp.mean((y - mu) ** 2, axis=-1, keepdims=True)
  o_ref[...] = (y - mu) / jnp.sqrt(var + 1e-5) * g_ref[...] + be_ref[...]


def _post(hd, agg_pair, cnt_hists, wl, bl, wr, g, be):
  return pl.pallas_call(
      _post_body,
      grid=(NB,),
      in_specs=[
          pl.BlockSpec((ROWBLK, D), lambda i: (i, 0)),
          pl.BlockSpec((1, ROWBLK, D), lambda i: (0, i, 0)),
          pl.BlockSpec((1, ROWBLK, D), lambda i: (1, i, 0)),
          pl.BlockSpec((ROWBLK, NW), lambda i: (i, 0)),
          pl.BlockSpec((D, D), lambda i: (0, 0)),
          pl.BlockSpec((1, D), lambda i: (0, 0)),
          pl.BlockSpec((D, D), lambda i: (0, 0)),
          pl.BlockSpec((1, D), lambda i: (0, 0)),
          pl.BlockSpec((1, D), lambda i: (0, 0)),
      ],
      out_specs=pl.BlockSpec((ROWBLK, D), lambda i: (i, 0)),
      out_shape=jax.ShapeDtypeStruct((N, D), jnp.float32),
  )(hd, agg_pair, agg_pair, cnt_hists, wl, bl.reshape(1, D), wr,
    g.reshape(1, D), be.reshape(1, D))


def kernel(x_user, x_item, edge_index_user_item, edge_index_item_user,
           W_user, b_user, W_item, b_item,
           Wl_ui, bl_ui, Wr_ui, Wl_iu, bl_iu, Wr_iu,
           ln_g_user, ln_b_user, ln_g_item, ln_b_item):
  h_user = _input_transform(x_user, W_user, b_user)
  h_item = _input_transform(x_item, W_item, b_item)

  agg, cnt = _sc_segment_sums(
      h_user, h_item,
      edge_index_user_item[0], edge_index_user_item[1],
      edge_index_item_user[0], edge_index_item_user[1])

  cnt_t = jnp.transpose(cnt, (0, 2, 1))
  out_item = _post(h_item, agg[0], cnt_t[0], Wl_ui, bl_ui, Wr_ui,
                   ln_g_item, ln_b_item)
  out_user = _post(h_user, agg[1], cnt_t[1], Wl_iu, bl_iu, Wr_iu,
                   ln_g_user, ln_b_user)
  return (out_user, out_item)


# speedup vs baseline: 3.8211x; 1.1157x over previous
import functools

import jax
import jax.numpy as jnp
from jax import lax
from jax.experimental import pallas as pl
from jax.experimental.pallas import tpu as pltpu
from jax.experimental.pallas import tpu_sc as plsc

N = 10000
E = 320000
D = 128
NC = 2
NS = 16
NW = NC * NS
K = 128
NPAD = 10240
RPT = NPAD // NS
EPT = NPAD
EPAD = NW * EPT
NCHUNK = EPT // K
NBUF = 2
NI = 8
ILA = 6

ROWBLK = 1000
NB = N // ROWBLK


def _sc_segment_sums(h_user, h_item, src0, dst0, src1, dst1):
  zrows = jnp.zeros((RPT, D), jnp.float32)
  zhist = jnp.zeros((NPAD,), jnp.float32)
  pad_s = jnp.arange(EPAD - E, dtype=src0.dtype) % N
  pad_d = (N + jnp.arange(EPAD - E, dtype=dst0.dtype) % (NPAD - N))
  src0 = jnp.concatenate([src0, pad_s])
  dst0 = jnp.concatenate([dst0, pad_d])
  src1 = jnp.concatenate([src1, pad_s])
  dst1 = jnp.concatenate([dst1, pad_d])

  mesh = plsc.VectorSubcoreMesh(core_axis_name="c", subcore_axis_name="s")

  @functools.partial(
      pl.kernel,
      out_type=(
          jax.ShapeDtypeStruct((2 * NC * NPAD, D), jnp.float32),
          jax.ShapeDtypeStruct((2 * NW * NPAD,), jnp.float32),
      ),
      mesh=mesh,
      compiler_params=pltpu.CompilerParams(needs_layout_passes=False),
      scratch_types=[
          pltpu.VMEM((NI, K), jnp.int32),
          pltpu.VMEM((NI, K), jnp.int32),
          pltpu.VMEM((NBUF, K, D), jnp.float32),
          pltpu.VMEM((NPAD,), jnp.float32),
          pltpu.VMEM_SHARED((NPAD, D), jnp.float32),
          [pltpu.SemaphoreType.DMA] * NI,
          [pltpu.SemaphoreType.DMA] * NBUF,
          [pltpu.SemaphoreType.DMA] * NBUF,
      ],
  )
  def seg(hu, hi, s0, d0, s1, d1, zr, zh, agg_out, cnt_out,
          idx_s, idx_d, rows, hist, acc, sem_i, sem_g, sem_s):
    c = lax.axis_index("c")
    s = lax.axis_index("s")
    wid = c * NS + s
    rowbase = pl.multiple_of(s * RPT, 8)
    ebase = wid * EPT
    ones16 = jnp.ones((16,), jnp.float32)

    def zero_owned():
      pltpu.sync_copy(zr, acc.at[pl.ds(rowbase, RPT)])
      pltpu.sync_copy(zh, hist)

    zero_owned()
    plsc.subcore_barrier()

    def do_rel(rel, table, src_hbm, dst_hbm):
      def fire_idx(ch, bi):
        eoff = pl.multiple_of(ebase + ch * K, 8)
        pltpu.async_copy(src_hbm.at[pl.ds(eoff, K)], idx_s.at[bi], sem_i[bi])
        pltpu.async_copy(dst_hbm.at[pl.ds(eoff, K)], idx_d.at[bi], sem_i[bi])

      def wait_idx(bi):
        pltpu.make_async_copy(src_hbm.at[pl.ds(0, K)], idx_s.at[bi],
                              sem_i[bi]).wait()
        pltpu.make_async_copy(src_hbm.at[pl.ds(0, K)], idx_d.at[bi],
                              sem_i[bi]).wait()

      def fire_gather(bi, b):
        pltpu.async_copy(table.at[idx_s.at[bi]], rows.at[b], sem_g[b])

      for ch in range(ILA):
        fire_idx(ch, ch % NI)
      for ch in range(NBUF):
        wait_idx(ch % NI)
        fire_gather(ch % NI, ch % NBUF)

      def outer(g, carry):
        for u in range(NI):
          ch = NI * g + u
          b = u % NBUF
          bi = u
          @pl.when(ch + ILA < NCHUNK)
          def _():
            fire_idx(ch + ILA, (u + ILA) % NI)
          pltpu.make_async_copy(table.at[pl.ds(0, K)], rows.at[b],
                                sem_g[b]).wait()
          pltpu.async_copy(rows.at[b], acc.at[idx_d.at[bi]], sem_s[b],
                           add=True)
          for t in range(K // 16):
            plsc.addupdate_scatter(hist, [idx_d[bi, pl.ds(t * 16, 16)]],
                                   ones16)
          @pl.when(ch + NBUF < NCHUNK)
          def _():
            pltpu.make_async_copy(table.at[pl.ds(0, K)], rows.at[b],
                                  sem_s[b]).wait()
            wait_idx((u + NBUF) % NI)
            fire_gather((u + NBUF) % NI, b)
        return carry
      lax.fori_loop(0, NCHUNK // NI, outer, 0)
      for b in range(NBUF):
        pltpu.make_async_copy(table.at[pl.ds(0, K)], rows.at[b],
                              sem_s[b]).wait()
      plsc.subcore_barrier()
      obase = pl.multiple_of((rel * NC + c) * NPAD + rowbase, 8)
      pltpu.sync_copy(acc.at[pl.ds(rowbase, RPT)],
                      agg_out.at[pl.ds(obase, RPT)])
      hbase = pl.multiple_of((rel * NW + wid) * NPAD, 8)
      pltpu.sync_copy(hist, cnt_out.at[pl.ds(hbase, NPAD)])

    do_rel(0, hu, s0, d0)
    zero_owned()
    plsc.subcore_barrier()
    do_rel(1, hi, s1, d1)

  agg, cnt = seg(h_user, h_item, src0, dst0, src1, dst1, zrows, zhist)
  return (agg.reshape(2, NC, NPAD, D), cnt.reshape(2, NW, NPAD))


def _lin_body(xu_ref, xi_ref, wu_ref, bu_ref, wi_ref, bi_ref,
              ou_ref, oi_ref):
  ou_ref[...] = lax.dot_general(
      xu_ref[...], wu_ref[...], (((1,), (1,)), ((), ())),
      preferred_element_type=jnp.float32) + bu_ref[...]
  oi_ref[...] = lax.dot_general(
      xi_ref[...], wi_ref[...], (((1,), (1,)), ((), ())),
      preferred_element_type=jnp.float32) + bi_ref[...]


def _input_transform(xu, xi, wu, bu, wi, bi):
  row = pl.BlockSpec((ROWBLK, D), lambda i: (i, 0))
  full = pl.BlockSpec((D, D), lambda i: (0, 0))
  vec = pl.BlockSpec((1, D), lambda i: (0, 0))
  return pl.pallas_call(
      _lin_body,
      grid=(NB,),
      in_specs=[row, row, full, vec, full, vec],
      out_specs=(row, row),
      out_shape=(jax.ShapeDtypeStruct((N, D), jnp.float32),
                 jax.ShapeDtypeStruct((N, D), jnp.float32)),
  )(xu, xi, wu, bu.reshape(1, D), wi, bi.reshape(1, D))


def _post_one(hd, a0, a1, c, wl, bl, wr, g, be):
  agg = a0[0, 0] + a1[0, 0]
  cnt = jnp.sum(c[0], axis=1, keepdims=True)
  mean = agg / jnp.maximum(cnt, 1.0)
  out = (lax.dot_general(mean, wl, (((1,), (1,)), ((), ())),
                         preferred_element_type=jnp.float32)
         + bl
         + lax.dot_general(hd, wr, (((1,), (1,)), ((), ())),
                           preferred_element_type=jnp.float32))
  nrm = jnp.sqrt(jnp.sum(out * out, axis=-1, keepdims=True))
  conv = out / jnp.maximum(nrm, 1e-12)
  y = hd + conv
  mu = jnp.mean(y, axis=-1, keepdims=True)
  var = jnp.mean((y - mu) ** 2, axis=-1, keepdims=True)
  return (y - mu) / jnp.sqrt(var + 1e-5) * g + be


def _post_body(hu_ref, hi_ref, au0_ref, au1_ref, ai0_ref, ai1_ref,
               cu_ref, ci_ref, wlu_ref, blu_ref, wru_ref, gu_ref, beu_ref,
               wli_ref, bli_ref, wri_ref, gi_ref, bei_ref,
               ou_ref, oi_ref):
  ou_ref[...] = _post_one(hu_ref[...], au0_ref, au1_ref, cu_ref,
                          wlu_ref[...], blu_ref[...], wru_ref[...],
                          gu_ref[...], beu_ref[...])
  oi_ref[...] = _post_one(hi_ref[...], ai0_ref, ai1_ref, ci_ref,
                          wli_ref[...], bli_ref[...], wri_ref[...],
                          gi_ref[...], bei_ref[...])


def _post(hu, hi, agg, cnt_t, wlu, blu, wru, gu, beu,
          wli, bli, wri, gi, bei):
  row = pl.BlockSpec((ROWBLK, D), lambda i: (i, 0))
  full = pl.BlockSpec((D, D), lambda i: (0, 0))
  vec = pl.BlockSpec((1, D), lambda i: (0, 0))
  out_sds = jax.ShapeDtypeStruct((N, D), jnp.float32)
  return pl.pallas_call(
      _post_body,
      grid=(NB,),
      in_specs=[
          row, row,
          pl.BlockSpec((1, 1, ROWBLK, D), lambda i: (1, 0, i, 0)),
          pl.BlockSpec((1, 1, ROWBLK, D), lambda i: (1, 1, i, 0)),
          pl.BlockSpec((1, 1, ROWBLK, D), lambda i: (0, 0, i, 0)),
          pl.BlockSpec((1, 1, ROWBLK, D), lambda i: (0, 1, i, 0)),
          pl.BlockSpec((1, ROWBLK, NW), lambda i: (1, i, 0)),
          pl.BlockSpec((1, ROWBLK, NW), lambda i: (0, i, 0)),
          full, vec, full, vec, vec,
          full, vec, full, vec, vec,
      ],
      out_specs=(row, row),
      out_shape=(out_sds, out_sds),
  )(hu, hi, agg, agg, agg, agg, cnt_t, cnt_t,
    wlu, blu.reshape(1, D), wru, gu.reshape(1, D), beu.reshape(1, D),
    wli, bli.reshape(1, D), wri, gi.reshape(1, D), bei.reshape(1, D))


def kernel(x_user, x_item, edge_index_user_item, edge_index_item_user,
           W_user, b_user, W_item, b_item,
           Wl_ui, bl_ui, Wr_ui, Wl_iu, bl_iu, Wr_iu,
           ln_g_user, ln_b_user, ln_g_item, ln_b_item):
  h_user, h_item = _input_transform(x_user, x_item, W_user, b_user,
                                    W_item, b_item)

  agg, cnt = _sc_segment_sums(
      h_user, h_item,
      edge_index_user_item[0], edge_index_user_item[1],
      edge_index_item_user[0], edge_index_item_user[1])

  cnt_t = jnp.transpose(cnt, (0, 2, 1))
  out_user, out_item = _post(
      h_user, h_item, agg, cnt_t,
      Wl_iu, bl_iu, Wr_iu, ln_g_user, ln_b_user,
      Wl_ui, bl_ui, Wr_ui, ln_g_item, ln_b_item)
  return (out_user, out_item)


# speedup vs baseline: 3.8847x; 1.0166x over previous
import functools

import jax
import jax.numpy as jnp
from jax import lax
from jax.experimental import pallas as pl
from jax.experimental.pallas import tpu as pltpu
from jax.experimental.pallas import tpu_sc as plsc

N = 10000
E = 320000
D = 128
NC = 2
NS = 16
NW = NC * NS
K = 128
NPAD = 10240
RPT = NPAD // NS
EPT = NPAD
EPAD = NW * EPT
NCHUNK = EPT // K
NBUF = 2
NI = 8
ILA = 6

ROWBLK = 2000
NB = N // ROWBLK


def _sc_segment_sums(h_user, h_item, src0, dst0, src1, dst1):
  zrows = jnp.zeros((RPT, D), jnp.float32)
  zhist = jnp.zeros((NPAD,), jnp.float32)
  pad_s = jnp.arange(EPAD - E, dtype=src0.dtype) % N
  pad_d = (N + jnp.arange(EPAD - E, dtype=dst0.dtype) % (NPAD - N))
  src0 = jnp.concatenate([src0, pad_s])
  dst0 = jnp.concatenate([dst0, pad_d])
  src1 = jnp.concatenate([src1, pad_s])
  dst1 = jnp.concatenate([dst1, pad_d])

  mesh = plsc.VectorSubcoreMesh(core_axis_name="c", subcore_axis_name="s")

  @functools.partial(
      pl.kernel,
      out_type=(
          jax.ShapeDtypeStruct((2 * NC * NPAD, D), jnp.float32),
          jax.ShapeDtypeStruct((2 * NW * NPAD,), jnp.float32),
      ),
      mesh=mesh,
      compiler_params=pltpu.CompilerParams(needs_layout_passes=False),
      scratch_types=[
          pltpu.VMEM((NI, K), jnp.int32),
          pltpu.VMEM((NI, K), jnp.int32),
          pltpu.VMEM((NBUF, K, D), jnp.float32),
          pltpu.VMEM((NPAD,), jnp.float32),
          pltpu.VMEM_SHARED((NPAD, D), jnp.float32),
          [pltpu.SemaphoreType.DMA] * NI,
          [pltpu.SemaphoreType.DMA] * NBUF,
          [pltpu.SemaphoreType.DMA] * NBUF,
      ],
  )
  def seg(hu, hi, s0, d0, s1, d1, zr, zh, agg_out, cnt_out,
          idx_s, idx_d, rows, hist, acc, sem_i, sem_g, sem_s):
    c = lax.axis_index("c")
    s = lax.axis_index("s")
    wid = c * NS + s
    rowbase = pl.multiple_of(s * RPT, 8)
    ebase = wid * EPT
    ones16 = jnp.ones((16,), jnp.float32)

    def zero_owned():
      pltpu.sync_copy(zr, acc.at[pl.ds(rowbase, RPT)])
      pltpu.sync_copy(zh, hist)

    zero_owned()
    plsc.subcore_barrier()

    def do_rel(rel, table, src_hbm, dst_hbm):
      def fire_idx(ch, bi):
        eoff = pl.multiple_of(ebase + ch * K, 8)
        pltpu.async_copy(src_hbm.at[pl.ds(eoff, K)], idx_s.at[bi], sem_i[bi])
        pltpu.async_copy(dst_hbm.at[pl.ds(eoff, K)], idx_d.at[bi], sem_i[bi])

      def wait_idx(bi):
        pltpu.make_async_copy(src_hbm.at[pl.ds(0, K)], idx_s.at[bi],
                              sem_i[bi]).wait()
        pltpu.make_async_copy(src_hbm.at[pl.ds(0, K)], idx_d.at[bi],
                              sem_i[bi]).wait()

      def fire_gather(bi, b):
        pltpu.async_copy(table.at[idx_s.at[bi]], rows.at[b], sem_g[b])

      for ch in range(ILA):
        fire_idx(ch, ch % NI)
      for ch in range(NBUF):
        wait_idx(ch % NI)
        fire_gather(ch % NI, ch % NBUF)

      def outer(g, carry):
        for u in range(NI):
          ch = NI * g + u
          b = u % NBUF
          bi = u
          @pl.when(ch + ILA < NCHUNK)
          def _():
            fire_idx(ch + ILA, (u + ILA) % NI)
          pltpu.make_async_copy(table.at[pl.ds(0, K)], rows.at[b],
                                sem_g[b]).wait()
          pltpu.async_copy(rows.at[b], acc.at[idx_d.at[bi]], sem_s[b],
                           add=True)
          for t in range(K // 16):
            plsc.addupdate_scatter(hist, [idx_d[bi, pl.ds(t * 16, 16)]],
                                   ones16)
          @pl.when(ch + NBUF < NCHUNK)
          def _():
            pltpu.make_async_copy(table.at[pl.ds(0, K)], rows.at[b],
                                  sem_s[b]).wait()
            wait_idx((u + NBUF) % NI)
            fire_gather((u + NBUF) % NI, b)
        return carry
      lax.fori_loop(0, NCHUNK // NI, outer, 0)
      for b in range(NBUF):
        pltpu.make_async_copy(table.at[pl.ds(0, K)], rows.at[b],
                              sem_s[b]).wait()
      plsc.subcore_barrier()
      obase = pl.multiple_of((rel * NC + c) * NPAD + rowbase, 8)
      pltpu.sync_copy(acc.at[pl.ds(rowbase, RPT)],
                      agg_out.at[pl.ds(obase, RPT)])
      hbase = pl.multiple_of((rel * NW + wid) * NPAD, 8)
      pltpu.sync_copy(hist, cnt_out.at[pl.ds(hbase, NPAD)])

    do_rel(0, hu, s0, d0)
    zero_owned()
    plsc.subcore_barrier()
    do_rel(1, hi, s1, d1)

  agg, cnt = seg(h_user, h_item, src0, dst0, src1, dst1, zrows, zhist)
  return (agg.reshape(2, NC, NPAD, D), cnt.reshape(2, NW, NPAD))


def _lin_body(xu_ref, xi_ref, wu_ref, bu_ref, wi_ref, bi_ref,
              ou_ref, oi_ref):
  ou_ref[...] = lax.dot_general(
      xu_ref[...], wu_ref[...], (((1,), (1,)), ((), ())),
      preferred_element_type=jnp.float32) + bu_ref[...]
  oi_ref[...] = lax.dot_general(
      xi_ref[...], wi_ref[...], (((1,), (1,)), ((), ())),
      preferred_element_type=jnp.float32) + bi_ref[...]


def _input_transform(xu, xi, wu, bu, wi, bi):
  row = pl.BlockSpec((ROWBLK, D), lambda i: (i, 0))
  full = pl.BlockSpec((D, D), lambda i: (0, 0))
  vec = pl.BlockSpec((1, D), lambda i: (0, 0))
  return pl.pallas_call(
      _lin_body,
      grid=(NB,),
      in_specs=[row, row, full, vec, full, vec],
      out_specs=(row, row),
      out_shape=(jax.ShapeDtypeStruct((N, D), jnp.float32),
                 jax.ShapeDtypeStruct((N, D), jnp.float32)),
  )(xu, xi, wu, bu.reshape(1, D), wi, bi.reshape(1, D))


def _post_one(hd, a0, a1, c, wl, bl, wr, g, be):
  agg = a0[0, 0] + a1[0, 0]
  cnt = jnp.sum(c[0], axis=1, keepdims=True)
  mean = agg / jnp.maximum(cnt, 1.0)
  out = (lax.dot_general(mean, wl, (((1,), (1,)), ((), ())),
                         preferred_element_type=jnp.float32)
         + bl
         + lax.dot_general(hd, wr, (((1,), (1,)), ((), ())),
                           preferred_element_type=jnp.float32))
  nrm = jnp.sqrt(jnp.sum(out * out, axis=-1, keepdims=True))
  conv = out / jnp.maximum(nrm, 1e-12)
  y = hd + conv
  mu = jnp.mean(y, axis=-1, keepdims=True)
  var = jnp.mean((y - mu) ** 2, axis=-1, keepdims=True)
  return (y - mu) / jnp.sqrt(var + 1e-5) * g + be


def _post_body(hu_ref, hi_ref, au0_ref, au1_ref, ai0_ref, ai1_ref,
               cu_ref, ci_ref, wlu_ref, blu_ref, wru_ref, gu_ref, beu_ref,
               wli_ref, bli_ref, wri_ref, gi_ref, bei_ref,
               ou_ref, oi_ref):
  ou_ref[...] = _post_one(hu_ref[...], au0_ref, au1_ref, cu_ref,
                          wlu_ref[...], blu_ref[...], wru_ref[...],
                          gu_ref[...], beu_ref[...])
  oi_ref[...] = _post_one(hi_ref[...], ai0_ref, ai1_ref, ci_ref,
                          wli_ref[...], bli_ref[...], wri_ref[...],
                          gi_ref[...], bei_ref[...])


def _post(hu, hi, agg, cnt_t, wlu, blu, wru, gu, beu,
          wli, bli, wri, gi, bei):
  row = pl.BlockSpec((ROWBLK, D), lambda i: (i, 0))
  full = pl.BlockSpec((D, D), lambda i: (0, 0))
  vec = pl.BlockSpec((1, D), lambda i: (0, 0))
  out_sds = jax.ShapeDtypeStruct((N, D), jnp.float32)
  return pl.pallas_call(
      _post_body,
      grid=(NB,),
      in_specs=[
          row, row,
          pl.BlockSpec((1, 1, ROWBLK, D), lambda i: (1, 0, i, 0)),
          pl.BlockSpec((1, 1, ROWBLK, D), lambda i: (1, 1, i, 0)),
          pl.BlockSpec((1, 1, ROWBLK, D), lambda i: (0, 0, i, 0)),
          pl.BlockSpec((1, 1, ROWBLK, D), lambda i: (0, 1, i, 0)),
          pl.BlockSpec((1, ROWBLK, NW), lambda i: (1, i, 0)),
          pl.BlockSpec((1, ROWBLK, NW), lambda i: (0, i, 0)),
          full, vec, full, vec, vec,
          full, vec, full, vec, vec,
      ],
      out_specs=(row, row),
      out_shape=(out_sds, out_sds),
  )(hu, hi, agg, agg, agg, agg, cnt_t, cnt_t,
    wlu, blu.reshape(1, D), wru, gu.reshape(1, D), beu.reshape(1, D),
    wli, bli.reshape(1, D), wri, gi.reshape(1, D), bei.reshape(1, D))


def kernel(x_user, x_item, edge_index_user_item, edge_index_item_user,
           W_user, b_user, W_item, b_item,
           Wl_ui, bl_ui, Wr_ui, Wl_iu, bl_iu, Wr_iu,
           ln_g_user, ln_b_user, ln_g_item, ln_b_item):
  h_user, h_item = _input_transform(x_user, x_item, W_user, b_user,
                                    W_item, b_item)

  agg, cnt = _sc_segment_sums(
      h_user, h_item,
      edge_index_user_item[0], edge_index_user_item[1],
      edge_index_item_user[0], edge_index_item_user[1])

  cnt_t = jnp.transpose(cnt, (0, 2, 1))
  out_user, out_item = _post(
      h_user, h_item, agg, cnt_t,
      Wl_iu, bl_iu, Wr_iu, ln_g_user, ln_b_user,
      Wl_ui, bl_ui, Wr_ui, ln_g_item, ln_b_item)
  return (out_user, out_item)


# speedup vs baseline: 3.9550x; 1.0181x over previous
import functools

import jax
import jax.numpy as jnp
from jax import lax
from jax.experimental import pallas as pl
from jax.experimental.pallas import tpu as pltpu
from jax.experimental.pallas import tpu_sc as plsc

N = 10000
E = 320000
D = 128
NC = 2
NS = 16
NW = NC * NS
K = 128
NPAD = 10240
RPT = NPAD // NS
EPT = NPAD
EPAD = NW * EPT
NCHUNK = EPT // K
NBUF = 2
NI = 8
ILA = 6

ROWBLK = 2048
NB = -(-N // ROWBLK)


def _sc_segment_sums(h_user, h_item, src0, dst0, src1, dst1):
  zrows = jnp.zeros((RPT, D), jnp.float32)
  zhist = jnp.zeros((NPAD,), jnp.float32)
  pad_s = jnp.arange(EPAD - E, dtype=src0.dtype) % N
  pad_d = (N + jnp.arange(EPAD - E, dtype=dst0.dtype) % (NPAD - N))
  src0 = jnp.concatenate([src0, pad_s])
  dst0 = jnp.concatenate([dst0, pad_d])
  src1 = jnp.concatenate([src1, pad_s])
  dst1 = jnp.concatenate([dst1, pad_d])

  mesh = plsc.VectorSubcoreMesh(core_axis_name="c", subcore_axis_name="s")

  @functools.partial(
      pl.kernel,
      out_type=(
          jax.ShapeDtypeStruct((2 * NC * NPAD, D), jnp.float32),
          jax.ShapeDtypeStruct((2 * NW * NPAD,), jnp.float32),
      ),
      mesh=mesh,
      compiler_params=pltpu.CompilerParams(needs_layout_passes=False),
      scratch_types=[
          pltpu.VMEM((NI, K), jnp.int32),
          pltpu.VMEM((NI, K), jnp.int32),
          pltpu.VMEM((NBUF, K, D), jnp.float32),
          pltpu.VMEM((NPAD,), jnp.float32),
          pltpu.VMEM_SHARED((NPAD, D), jnp.float32),
          [pltpu.SemaphoreType.DMA] * NI,
          [pltpu.SemaphoreType.DMA] * NBUF,
          [pltpu.SemaphoreType.DMA] * NBUF,
      ],
  )
  def seg(hu, hi, s0, d0, s1, d1, zr, zh, agg_out, cnt_out,
          idx_s, idx_d, rows, hist, acc, sem_i, sem_g, sem_s):
    c = lax.axis_index("c")
    s = lax.axis_index("s")
    wid = c * NS + s
    rowbase = pl.multiple_of(s * RPT, 8)
    ebase = wid * EPT
    ones16 = jnp.ones((16,), jnp.float32)

    def zero_owned():
      pltpu.sync_copy(zr, acc.at[pl.ds(rowbase, RPT)])
      pltpu.sync_copy(zh, hist)

    zero_owned()
    plsc.subcore_barrier()

    def do_rel(rel, table, src_hbm, dst_hbm):
      def fire_idx(ch, bi):
        eoff = pl.multiple_of(ebase + ch * K, 8)
        pltpu.async_copy(src_hbm.at[pl.ds(eoff, K)], idx_s.at[bi], sem_i[bi])
        pltpu.async_copy(dst_hbm.at[pl.ds(eoff, K)], idx_d.at[bi], sem_i[bi])

      def wait_idx(bi):
        pltpu.make_async_copy(src_hbm.at[pl.ds(0, K)], idx_s.at[bi],
                              sem_i[bi]).wait()
        pltpu.make_async_copy(src_hbm.at[pl.ds(0, K)], idx_d.at[bi],
                              sem_i[bi]).wait()

      def fire_gather(bi, b):
        pltpu.async_copy(table.at[idx_s.at[bi]], rows.at[b], sem_g[b])

      for ch in range(ILA):
        fire_idx(ch, ch % NI)
      for ch in range(NBUF):
        wait_idx(ch % NI)
        fire_gather(ch % NI, ch % NBUF)

      def outer(g, carry):
        for u in range(NI):
          ch = NI * g + u
          b = u % NBUF
          bi = u
          @pl.when(ch + ILA < NCHUNK)
          def _():
            fire_idx(ch + ILA, (u + ILA) % NI)
          pltpu.make_async_copy(table.at[pl.ds(0, K)], rows.at[b],
                                sem_g[b]).wait()
          pltpu.async_copy(rows.at[b], acc.at[idx_d.at[bi]], sem_s[b],
                           add=True)
          for t in range(K // 16):
            plsc.addupdate_scatter(hist, [idx_d[bi, pl.ds(t * 16, 16)]],
                                   ones16)
          @pl.when(ch + NBUF < NCHUNK)
          def _():
            pltpu.make_async_copy(table.at[pl.ds(0, K)], rows.at[b],
                                  sem_s[b]).wait()
            wait_idx((u + NBUF) % NI)
            fire_gather((u + NBUF) % NI, b)
        return carry
      lax.fori_loop(0, NCHUNK // NI, outer, 0)
      for b in range(NBUF):
        pltpu.make_async_copy(table.at[pl.ds(0, K)], rows.at[b],
                              sem_s[b]).wait()
      plsc.subcore_barrier()
      obase = pl.multiple_of((rel * NC + c) * NPAD + rowbase, 8)
      pltpu.sync_copy(acc.at[pl.ds(rowbase, RPT)],
                      agg_out.at[pl.ds(obase, RPT)])
      hbase = pl.multiple_of((rel * NW + wid) * NPAD, 8)
      pltpu.sync_copy(hist, cnt_out.at[pl.ds(hbase, NPAD)])

    do_rel(0, hu, s0, d0)
    zero_owned()
    plsc.subcore_barrier()
    do_rel(1, hi, s1, d1)

  agg, cnt = seg(h_user, h_item, src0, dst0, src1, dst1, zrows, zhist)
  return (agg.reshape(2, NC, NPAD, D), cnt.reshape(2, NW, NPAD))


def _lin_body(xu_ref, xi_ref, wu_ref, bu_ref, wi_ref, bi_ref,
              ou_ref, oi_ref):
  ou_ref[...] = lax.dot_general(
      xu_ref[...], wu_ref[...], (((1,), (1,)), ((), ())),
      preferred_element_type=jnp.float32) + bu_ref[...]
  oi_ref[...] = lax.dot_general(
      xi_ref[...], wi_ref[...], (((1,), (1,)), ((), ())),
      preferred_element_type=jnp.float32) + bi_ref[...]


def _input_transform(xu, xi, wu, bu, wi, bi):
  row = pl.BlockSpec((ROWBLK, D), lambda i: (i, 0))
  full = pl.BlockSpec((D, D), lambda i: (0, 0))
  vec = pl.BlockSpec((1, D), lambda i: (0, 0))
  return pl.pallas_call(
      _lin_body,
      grid=(NB,),
      in_specs=[row, row, full, vec, full, vec],
      out_specs=(row, row),
      out_shape=(jax.ShapeDtypeStruct((N, D), jnp.float32),
                 jax.ShapeDtypeStruct((N, D), jnp.float32)),
  )(xu, xi, wu, bu.reshape(1, D), wi, bi.reshape(1, D))


def _post_one(hd, a0, a1, c, wl, bl, wr, g, be):
  agg = a0[0, 0] + a1[0, 0]
  cnt = jnp.sum(c[0], axis=0)[:, None]
  mean = agg / jnp.maximum(cnt, 1.0)
  out = (lax.dot_general(mean, wl, (((1,), (1,)), ((), ())),
                         preferred_element_type=jnp.float32)
         + bl
         + lax.dot_general(hd, wr, (((1,), (1,)), ((), ())),
                           preferred_element_type=jnp.float32))
  nrm = jnp.sqrt(jnp.sum(out * out, axis=-1, keepdims=True))
  conv = out / jnp.maximum(nrm, 1e-12)
  y = hd + conv
  mu = jnp.mean(y, axis=-1, keepdims=True)
  var = jnp.mean((y - mu) ** 2, axis=-1, keepdims=True)
  return (y - mu) / jnp.sqrt(var + 1e-5) * g + be


def _post_body(hu_ref, hi_ref, au0_ref, au1_ref, ai0_ref, ai1_ref,
               cu_ref, ci_ref, wlu_ref, blu_ref, wru_ref, gu_ref, beu_ref,
               wli_ref, bli_ref, wri_ref, gi_ref, bei_ref,
               ou_ref, oi_ref):
  ou_ref[...] = _post_one(hu_ref[...], au0_ref, au1_ref, cu_ref,
                          wlu_ref[...], blu_ref[...], wru_ref[...],
                          gu_ref[...], beu_ref[...])
  oi_ref[...] = _post_one(hi_ref[...], ai0_ref, ai1_ref, ci_ref,
                          wli_ref[...], bli_ref[...], wri_ref[...],
                          gi_ref[...], bei_ref[...])


def _post(hu, hi, agg, cnt, wlu, blu, wru, gu, beu,
          wli, bli, wri, gi, bei):
  row = pl.BlockSpec((ROWBLK, D), lambda i: (i, 0))
  full = pl.BlockSpec((D, D), lambda i: (0, 0))
  vec = pl.BlockSpec((1, D), lambda i: (0, 0))
  out_sds = jax.ShapeDtypeStruct((N, D), jnp.float32)
  return pl.pallas_call(
      _post_body,
      grid=(NB,),
      in_specs=[
          row, row,
          pl.BlockSpec((1, 1, ROWBLK, D), lambda i: (1, 0, i, 0)),
          pl.BlockSpec((1, 1, ROWBLK, D), lambda i: (1, 1, i, 0)),
          pl.BlockSpec((1, 1, ROWBLK, D), lambda i: (0, 0, i, 0)),
          pl.BlockSpec((1, 1, ROWBLK, D), lambda i: (0, 1, i, 0)),
          pl.BlockSpec((1, NW, ROWBLK), lambda i: (1, 0, i)),
          pl.BlockSpec((1, NW, ROWBLK), lambda i: (0, 0, i)),
          full, vec, full, vec, vec,
          full, vec, full, vec, vec,
      ],
      out_specs=(row, row),
      out_shape=(out_sds, out_sds),
  )(hu, hi, agg, agg, agg, agg, cnt, cnt,
    wlu, blu.reshape(1, D), wru, gu.reshape(1, D), beu.reshape(1, D),
    wli, bli.reshape(1, D), wri, gi.reshape(1, D), bei.reshape(1, D))


def kernel(x_user, x_item, edge_index_user_item, edge_index_item_user,
           W_user, b_user, W_item, b_item,
           Wl_ui, bl_ui, Wr_ui, Wl_iu, bl_iu, Wr_iu,
           ln_g_user, ln_b_user, ln_g_item, ln_b_item):
  h_user, h_item = _input_transform(x_user, x_item, W_user, b_user,
                                    W_item, b_item)

  agg, cnt = _sc_segment_sums(
      h_user, h_item,
      edge_index_user_item[0], edge_index_user_item[1],
      edge_index_item_user[0], edge_index_item_user[1])

  out_user, out_item = _post(
      h_user, h_item, agg, cnt,
      Wl_iu, bl_iu, Wr_iu, ln_g_user, ln_b_user,
      Wl_ui, bl_ui, Wr_ui, ln_g_item, ln_b_item)
  return (out_user, out_item)
